# baseline (speedup 1.0000x reference)
import functools

import jax
import jax.numpy as jnp
from jax import lax
from jax.experimental import pallas as pl
from jax.experimental.pallas import tpu as pltpu
from jax.experimental.pallas import tpu_sc as plsc


def _scatter_rows(tok_k, tok_v, slot_mapping, k_ref, v_ref):
    n_tok, n_heads, head_dim = tok_k.shape
    mesh = plsc.VectorSubcoreMesh(core_axis_name="c", subcore_axis_name="s")

    @functools.partial(
        pl.kernel,
        mesh=mesh,
        out_type=(),
        scratch_types=[
            pltpu.VMEM((n_tok,), jnp.int32),
        ],
    )
    def body(tok_k_hbm, tok_v_hbm, slot_hbm, kc, vc, idx_v):
        wid = lax.axis_index("s") * 2 + lax.axis_index("c")

        def scatter_all(tok_hbm, cache):
            pltpu.sync_copy(slot_hbm, idx_v)
            for chunk in range(n_tok // 16):
                vec = idx_v[pl.ds(chunk * 16, 16)]
                for lane in range(16):
                    i = chunk * 16 + lane
                    pltpu.sync_copy(tok_hbm.at[i], cache.at[vec[lane]])

        @pl.when(wid == 0)
        def _():
            scatter_all(tok_k_hbm, kc)

        @pl.when(wid == 1)
        def _():
            scatter_all(tok_v_hbm, vc)

    body(tok_k, tok_v, slot_mapping, k_ref, v_ref)


def kernel(pos_ids, k_val, v_val, slot_mapping, batch_idx, k_cache, v_cache):
    B, H, S, D = k_val.shape
    tok_k = jnp.transpose(k_val, (0, 2, 1, 3)).reshape(B * S, H, D)
    tok_v = jnp.transpose(v_val, (0, 2, 1, 3)).reshape(B * S, H, D)
    k_ref = jax.new_ref(k_cache)
    v_ref = jax.new_ref(v_cache)
    _scatter_rows(tok_k, tok_v, slot_mapping, k_ref, v_ref)
    return k_ref[...], v_ref[...]
